# baseline (speedup 1.0000x reference)
import jax
import jax.numpy as jnp
from jax.experimental import pallas as pl

N_TOK = 16384
D = 10
NB = 4
BN = N_TOK // NB


def _body(xt_ref, r_ref, w1_ref, b1_ref, w2_ref, b2_ref, out_ref):
    xt = xt_ref[...]
    m = (r_ref[...] == 0).reshape(1, BN)
    wc = jnp.concatenate([w1_ref[...], w2_ref[...]], axis=0)
    yb = jax.lax.dot(wc, xt, preferred_element_type=jnp.float32)
    y1 = yb[:D] + b1_ref[...].reshape(D, 1)
    y2 = yb[D:] + b2_ref[...].reshape(D, 1)
    out_ref[...] = jnp.where(m, y1, y2)


def kernel(x, route, W1, b1, W2, b2):
    xt = x.T
    outt = pl.pallas_call(
        _body,
        grid=(NB,),
        in_specs=[
            pl.BlockSpec((D, BN), lambda i: (0, i)),
            pl.BlockSpec((BN,), lambda i: (i,)),
            pl.BlockSpec((D, D), lambda i: (0, 0)),
            pl.BlockSpec((D,), lambda i: (0,)),
            pl.BlockSpec((D, D), lambda i: (0, 0)),
            pl.BlockSpec((D,), lambda i: (0,)),
        ],
        out_specs=pl.BlockSpec((D, BN), lambda i: (0, i)),
        out_shape=jax.ShapeDtypeStruct((D, N_TOK), jnp.float32),
    )(xt, route.astype(jnp.int32), W1, b1, W2, b2)
    return outt.T


# speedup vs baseline: 1.0992x; 1.0992x over previous
import jax
import jax.numpy as jnp
from jax.experimental import pallas as pl
from jax.experimental.pallas import tpu as pltpu

N_TOK = 16384
D = 10
NB = 2
BN = N_TOK // NB


def _body(xt_hbm, r_hbm, w1_ref, b1_ref, w2_ref, b2_ref, out_hbm,
          xb, rb, ob, isems, rsems, osems):
    wc = jnp.concatenate([w1_ref[...], w2_ref[...]], axis=0)
    b1 = b1_ref[...].reshape(D, 1)
    b2 = b2_ref[...].reshape(D, 1)

    def start_in(s):
        pltpu.async_copy(xt_hbm.at[:, pl.ds(s * BN, BN)], xb.at[s], isems.at[s])
        pltpu.async_copy(r_hbm.at[pl.ds(s * BN, BN)], rb.at[s], rsems.at[s])

    start_in(0)
    start_in(1)
    outcp = []
    for s in range(NB):
        pltpu.make_async_copy(xt_hbm.at[:, pl.ds(s * BN, BN)], xb.at[s],
                              isems.at[s]).wait()
        pltpu.make_async_copy(r_hbm.at[pl.ds(s * BN, BN)], rb.at[s],
                              rsems.at[s]).wait()
        yb = jax.lax.dot(wc, xb[s], preferred_element_type=jnp.float32)
        m = (rb[s] == 0).reshape(1, BN)
        ob[s] = jnp.where(m, yb[:D] + b1, yb[D:] + b2)
        cp = pltpu.make_async_copy(ob.at[s], out_hbm.at[:, pl.ds(s * BN, BN)],
                                   osems.at[s])
        cp.start()
        outcp.append(cp)
    for cp in outcp:
        cp.wait()


def kernel(x, route, W1, b1, W2, b2):
    xt = x.T
    outt = pl.pallas_call(
        _body,
        in_specs=[
            pl.BlockSpec(memory_space=pl.ANY),
            pl.BlockSpec(memory_space=pl.ANY),
            pl.BlockSpec((D, D), lambda: (0, 0)),
            pl.BlockSpec((D,), lambda: (0,)),
            pl.BlockSpec((D, D), lambda: (0, 0)),
            pl.BlockSpec((D,), lambda: (0,)),
        ],
        out_specs=pl.BlockSpec(memory_space=pl.ANY),
        out_shape=jax.ShapeDtypeStruct((D, N_TOK), jnp.float32),
        scratch_shapes=[
            pltpu.VMEM((NB, D, BN), jnp.float32),
            pltpu.VMEM((NB, BN), jnp.int32),
            pltpu.VMEM((NB, D, BN), jnp.float32),
            pltpu.SemaphoreType.DMA((NB,)),
            pltpu.SemaphoreType.DMA((NB,)),
            pltpu.SemaphoreType.DMA((NB,)),
        ],
    )(xt, route.astype(jnp.int32), W1, b1, W2, b2)
    return outt.T


# speedup vs baseline: 1.4051x; 1.2784x over previous
import jax
import jax.numpy as jnp
from jax.experimental import pallas as pl

N_TOK = 16384
D = 10
NB = 2
BN = N_TOK // NB


def _body(xt_ref, r_ref, w1_ref, b1_ref, w2_ref, b2_ref, out_ref):
    xt = xt_ref[...]
    m = (r_ref[...] == 0).reshape(1, BN)
    wc = jnp.concatenate([w1_ref[...], w2_ref[...]], axis=0)
    yb = jax.lax.dot(wc, xt, preferred_element_type=jnp.float32)
    y1 = yb[:D] + b1_ref[...].reshape(D, 1)
    y2 = yb[D:] + b2_ref[...].reshape(D, 1)
    out_ref[...] = jnp.where(m, y1, y2)


def kernel(x, route, W1, b1, W2, b2):
    xt = x.T
    outt = pl.pallas_call(
        _body,
        grid=(NB,),
        in_specs=[
            pl.BlockSpec((D, BN), lambda i: (0, i)),
            pl.BlockSpec((BN,), lambda i: (i,)),
            pl.BlockSpec((D, D), lambda i: (0, 0)),
            pl.BlockSpec((D,), lambda i: (0,)),
            pl.BlockSpec((D, D), lambda i: (0, 0)),
            pl.BlockSpec((D,), lambda i: (0,)),
        ],
        out_specs=pl.BlockSpec((D, BN), lambda i: (0, i)),
        out_shape=jax.ShapeDtypeStruct((D, N_TOK), jnp.float32),
    )(xt, route.astype(jnp.int32), W1, b1, W2, b2)
    return outt.T
